# Initial kernel scaffold; baseline (speedup 1.0000x reference)
import jax
import jax.numpy as jnp
from jax.experimental import pallas as pl


def kernel(x_enc, x_mark_enc, x_dec, x_mark_dec, params):
    raise NotImplementedError("write your pallas kernel here")


# speedup vs baseline: 1.7389x; 1.7389x over previous
import numpy as np
import jax
import jax.numpy as jnp
from jax import lax
from jax.experimental import pallas as pl
from jax.experimental.pallas import tpu as pltpu

B, SEQ_LEN, ENC_IN, PRED_LEN = 8, 512, 21, 96
D_MODEL, D_FF, D_STATE, D_CONV, EXPAND = 128, 256, 16, 4, 2
E_LAYERS, D_LAYERS = 2, 2
PATCH_LEN, STRIDE = 16, 8
PATCH_NUM = (SEQ_LEN - PATCH_LEN) // STRIDE + 2
EPS = 1e-5


def _pos_emb_np(n, d):
    pos = np.arange(n)[:, None].astype(np.float32)
    div = np.exp(np.arange(0, d, 2).astype(np.float32) * (-np.log(10000.0) / d))
    pe = np.zeros((n, d), np.float32)
    pe[:, 0::2] = np.sin(pos * div)
    pe[:, 1::2] = np.cos(pos * div)
    return pe


_POS = _pos_emb_np(PATCH_NUM, D_MODEL)


def _ln(x, g, b):
    m = jnp.mean(x, -1, keepdims=True)
    v = jnp.mean((x - m) ** 2, -1, keepdims=True)
    return (x - m) * lax.rsqrt(v + EPS) * g + b


def _mamba_ffn_kernel(L, di, ds, dc, dtr, dm, dff):
    def body(x_ref, inw_ref, inb_ref, cwt_ref, cb_ref, xpw_ref, dtw_ref,
             dtb_ref, at_ref, dvec_ref, ow_ref, ob_ref, l1g_ref, l1b_ref,
             f1w_ref, f1b_ref, f2w_ref, f2b_ref, l2g_ref, l2b_ref,
             o_ref, dls, dus, bs, cs, yss):
        x = x_ref[0]
        xz = jnp.dot(x, inw_ref[...], preferred_element_type=jnp.float32)
        xz = xz + inb_ref[...]
        xi = xz[:, :di]
        z = xz[:, di:]
        xp = jnp.concatenate([jnp.zeros((dc - 1, di), jnp.float32), xi], axis=0)
        xc = cb_ref[...]
        for k in range(dc):
            xc = xc + xp[k:k + L, :] * cwt_ref[k:k + 1, :]
        xi = xc * jax.nn.sigmoid(xc)
        dbc = jnp.dot(xi, xpw_ref[...], preferred_element_type=jnp.float32)
        dt = dbc[:, :dtr]
        delta = jnp.dot(dt, dtw_ref[...], preferred_element_type=jnp.float32)
        delta = jax.nn.softplus(delta + dtb_ref[...])
        dls[...] = delta
        dus[...] = delta * xi
        bs[...] = dbc[:, dtr:dtr + ds]
        cs[...] = dbc[:, dtr + ds:dtr + 2 * ds]
        at = at_ref[...]

        def step(t, h):
            dr = dls[pl.ds(t, 1), :]
            dur = dus[pl.ds(t, 1), :]
            br = bs[pl.ds(t, 1), :]
            cr = cs[pl.ds(t, 1), :]
            h = h * jnp.exp(dr * at) + lax.dot_general(
                br, dur, (((0,), (0,)), ((), ())),
                preferred_element_type=jnp.float32)
            yss[pl.ds(t, 1), :] = lax.dot_general(
                cr, h, (((1,), (0,)), ((), ())),
                preferred_element_type=jnp.float32)
            return h

        lax.fori_loop(0, L, step, jnp.zeros((ds, di), jnp.float32))

        y = yss[...] + xi * dvec_ref[...]
        y = y * (z * jax.nn.sigmoid(z))
        ym = jnp.dot(y, ow_ref[...], preferred_element_type=jnp.float32)
        ym = ym + ob_ref[...]
        x1 = _ln(x + ym, l1g_ref[...], l1b_ref[...])
        hf = jnp.dot(x1, f1w_ref[...], preferred_element_type=jnp.float32)
        hf = jnp.maximum(hf + f1b_ref[...], 0.0)
        hf = jnp.dot(hf, f2w_ref[...], preferred_element_type=jnp.float32)
        hf = hf + f2b_ref[...]
        o_ref[0] = _ln(x1 + hf, l2g_ref[...], l2b_ref[...])
    return body


def _mamba_ffn(x, p):
    Bz, L, dm = x.shape
    di = p['D'].shape[0]
    ds = p['A_log'].shape[1]
    dc = p['conv_w'].shape[-1]
    dtr = p['dt_w'].shape[1]
    dff = p['f1w'].shape[0]
    row = lambda v: v.reshape(1, -1)
    ws = [
        p['in_w'].T, row(p['in_b']),
        p['conv_w'].T, row(p['conv_b']),
        p['xproj_w'].T,
        p['dt_w'].T, row(p['dt_b']),
        (-jnp.exp(p['A_log'])).T,
        row(p['D']),
        p['out_w'].T, row(p['out_b']),
        row(p['ln1_g']), row(p['ln1_b']),
        p['f1w'].T, row(p['f1b']),
        p['f2w'].T, row(p['f2b']),
        row(p['ln2_g']), row(p['ln2_b']),
    ]
    wspec = [pl.BlockSpec(w.shape, lambda b, n=w.ndim: (0,) * n) for w in ws]
    f32 = jnp.float32
    return pl.pallas_call(
        _mamba_ffn_kernel(L, di, ds, dc, dtr, dm, dff),
        grid=(Bz,),
        in_specs=[pl.BlockSpec((1, L, dm), lambda b: (b, 0, 0))] + wspec,
        out_specs=pl.BlockSpec((1, L, dm), lambda b: (b, 0, 0)),
        out_shape=jax.ShapeDtypeStruct((Bz, L, dm), f32),
        scratch_shapes=[
            pltpu.VMEM((L, di), f32), pltpu.VMEM((L, di), f32),
            pltpu.VMEM((L, ds), f32), pltpu.VMEM((L, ds), f32),
            pltpu.VMEM((L, di), f32),
        ],
        compiler_params=pltpu.CompilerParams(
            dimension_semantics=('parallel',)),
    )(x, *ws)


def _tem_kernel(P, D):
    def body(x_ref, w0_ref, w1_ref, w2_ref, b_ref, g_ref, be_ref, o_ref):
        x = x_ref[0]
        zp = jnp.zeros((1, D), jnp.float32)
        xpad = jnp.concatenate([zp, x, zp], axis=0)
        c = b_ref[...]
        for k in range(3):
            c = c + jnp.dot(xpad[k:k + P, :], [w0_ref, w1_ref, w2_ref][k][...],
                            preferred_element_type=jnp.float32)
        c = _ln(c, g_ref[...], be_ref[...])
        o_ref[0] = c + x
    return body


def _tem(enc_pd, tp):
    Bz, N, P, D = enc_pd.shape
    x = enc_pd.reshape(Bz * N, P, D)
    ws = [tp['conv_w'][:, :, 0].T, tp['conv_w'][:, :, 1].T,
          tp['conv_w'][:, :, 2].T, tp['conv_b'].reshape(1, -1),
          tp['ln_g'].reshape(1, -1), tp['ln_b'].reshape(1, -1)]
    wspec = [pl.BlockSpec(w.shape, lambda b, n=w.ndim: (0,) * n) for w in ws]
    out = pl.pallas_call(
        _tem_kernel(P, D),
        grid=(Bz * N,),
        in_specs=[pl.BlockSpec((1, P, D), lambda b: (b, 0, 0))] + wspec,
        out_specs=pl.BlockSpec((1, P, D), lambda b: (b, 0, 0)),
        out_shape=jax.ShapeDtypeStruct((Bz * N, P, D), jnp.float32),
        compiler_params=pltpu.CompilerParams(
            dimension_semantics=('parallel',)),
    )(x, *ws)
    return out.reshape(Bz, N, P, D)


def kernel(x_enc, x_mark_enc, x_dec, x_mark_dec, params):
    p = params
    mean = jnp.mean(x_enc, 1, keepdims=True)
    std = jnp.sqrt(jnp.var(x_enc, 1, keepdims=True) + EPS)
    xn = (x_enc - mean) / std * p['revin_w'] + p['revin_b']

    emb = jnp.swapaxes(xn, 1, 2) @ p['emb_w'].T + p['emb_b']
    for lp in p['mamba1']:
        emb = _mamba_ffn(emb, lp)
    x_var = jnp.swapaxes(emb @ p['proj_w'].T + p['proj_b'], 1, 2)

    xp = jnp.swapaxes(xn, 1, 2)
    xp = jnp.concatenate(
        [xp, jnp.broadcast_to(xp[..., -1:], xp.shape[:-1] + (STRIDE,))], -1)
    idx = np.arange(PATCH_NUM)[:, None] * STRIDE + np.arange(PATCH_LEN)[None, :]
    patches = xp[..., idx]
    enc = patches @ p['val_w'].T + jnp.asarray(_POS)
    Bz, N, P, D = enc.shape
    for i in range(E_LAYERS):
        enc_t = _tem(enc, p['tem'][i])
        vp = p['var'][i]
        tokens = jnp.swapaxes(enc_t, -1, -2).reshape(Bz, P * N, D)
        v = _mamba_ffn(tokens, vp['mamba'])
        v = _ln(v.reshape(Bz, N, P, D), vp['ln_g'], vp['ln_b'])
        v = v + enc_t
        g, b = (p['ln2_g'], p['ln2_b']) if i == 0 else (p['ln3_g'], p['ln3_b'])
        enc = _ln(v, g, b)
    enc_dp = jnp.swapaxes(enc, -1, -2).reshape(Bz, N, D * P)
    x_patch = jnp.swapaxes(enc_dp @ p['head_w'].T + p['head_b'], 1, 2)

    out = x_var + x_patch
    out = (out - p['revin_b']) / (p['revin_w'] + EPS * EPS)
    return out * std + mean


# speedup vs baseline: 4.9531x; 2.8485x over previous
import numpy as np
import jax
import jax.numpy as jnp
from jax import lax
from jax.experimental import pallas as pl
from jax.experimental.pallas import tpu as pltpu

B, SEQ_LEN, ENC_IN, PRED_LEN = 8, 512, 21, 96
D_MODEL, D_FF, D_STATE, D_CONV, EXPAND = 128, 256, 16, 4, 2
E_LAYERS, D_LAYERS = 2, 2
PATCH_LEN, STRIDE = 16, 8
PATCH_NUM = (SEQ_LEN - PATCH_LEN) // STRIDE + 2
EPS = 1e-5


def _pos_emb_np(n, d):
    pos = np.arange(n)[:, None].astype(np.float32)
    div = np.exp(np.arange(0, d, 2).astype(np.float32) * (-np.log(10000.0) / d))
    pe = np.zeros((n, d), np.float32)
    pe[:, 0::2] = np.sin(pos * div)
    pe[:, 1::2] = np.cos(pos * div)
    return pe


_POS = _pos_emb_np(PATCH_NUM, D_MODEL)


def _ln(x, g, b):
    m = jnp.mean(x, -1, keepdims=True)
    v = jnp.mean((x - m) ** 2, -1, keepdims=True)
    return (x - m) * lax.rsqrt(v + EPS) * g + b


def _mamba_ffn_kernel(L, T, di, ds, dc, dtr, dm, dff):
    def body(x_ref, inw_ref, inb_ref, cwt_ref, cb_ref, xpw_ref, dtw_ref,
             dtb_ref, at_ref, dvec_ref, ow_ref, ob_ref, l1g_ref, l1b_ref,
             f1w_ref, f1b_ref, f2w_ref, f2b_ref, l2g_ref, l2b_ref,
             o_ref, dls, dus, bs, cs, yss, es, fs, ccs):
        x = x_ref[0]
        xz = jnp.dot(x, inw_ref[...], preferred_element_type=jnp.float32)
        xz = xz + inb_ref[...]
        xi = xz[:, :di]
        z = xz[:, di:]
        xp = jnp.concatenate([jnp.zeros((dc - 1, di), jnp.float32), xi], axis=0)
        xc = cb_ref[...]
        for k in range(dc):
            xc = xc + xp[k:k + L, :] * cwt_ref[k:k + 1, :]
        xi = xc * jax.nn.sigmoid(xc)
        dbc = jnp.dot(xi, xpw_ref[...], preferred_element_type=jnp.float32)
        dt = dbc[:, :dtr]
        delta = jnp.dot(dt, dtw_ref[...], preferred_element_type=jnp.float32)
        delta = jax.nn.softplus(delta + dtb_ref[...])
        dls[...] = delta
        dus[...] = delta * xi
        bs[...] = dbc[:, dtr:dtr + ds]
        cs[...] = dbc[:, dtr + ds:dtr + 2 * ds]
        at = at_ref[...]

        def chunk(c, h):
            base = c * T
            dl = dls[pl.ds(base, T), :]
            du = dus[pl.ds(base, T), :]
            bc = bs[pl.ds(base, T), :]
            cc = cs[pl.ds(base, T), :]
            es[...] = jnp.exp(dl[:, None, :] * at[None, :, :])
            fs[...] = du[:, None, :] * bc[:, :, None]
            ccs[...] = cc[:, :, None]

            def step(t, h):
                e = es[pl.ds(t, 1)][0]
                f = fs[pl.ds(t, 1)][0]
                cl = ccs[pl.ds(t, 1)][0]
                h = h * e + f
                yss[pl.ds(base + t, 1), :] = jnp.sum(
                    h * cl, axis=0, keepdims=True)
                return h

            return lax.fori_loop(0, T, step, h)

        lax.fori_loop(0, L // T, chunk, jnp.zeros((ds, di), jnp.float32))

        y = yss[...] + xi * dvec_ref[...]
        y = y * (z * jax.nn.sigmoid(z))
        ym = jnp.dot(y, ow_ref[...], preferred_element_type=jnp.float32)
        ym = ym + ob_ref[...]
        x1 = _ln(x + ym, l1g_ref[...], l1b_ref[...])
        hf = jnp.dot(x1, f1w_ref[...], preferred_element_type=jnp.float32)
        hf = jnp.maximum(hf + f1b_ref[...], 0.0)
        hf = jnp.dot(hf, f2w_ref[...], preferred_element_type=jnp.float32)
        hf = hf + f2b_ref[...]
        o_ref[0] = _ln(x1 + hf, l2g_ref[...], l2b_ref[...])
    return body


def _mamba_ffn(x, p):
    Bz, L, dm = x.shape
    di = p['D'].shape[0]
    ds = p['A_log'].shape[1]
    dc = p['conv_w'].shape[-1]
    dtr = p['dt_w'].shape[1]
    dff = p['f1w'].shape[0]
    row = lambda v: v.reshape(1, -1)
    ws = [
        p['in_w'].T, row(p['in_b']),
        p['conv_w'].T, row(p['conv_b']),
        p['xproj_w'].T,
        p['dt_w'].T, row(p['dt_b']),
        (-jnp.exp(p['A_log'])).T,
        row(p['D']),
        p['out_w'].T, row(p['out_b']),
        row(p['ln1_g']), row(p['ln1_b']),
        p['f1w'].T, row(p['f1b']),
        p['f2w'].T, row(p['f2b']),
        row(p['ln2_g']), row(p['ln2_b']),
    ]
    wspec = [pl.BlockSpec(w.shape, lambda b, n=w.ndim: (0,) * n) for w in ws]
    f32 = jnp.float32
    T = 64 if L % 64 == 0 else L
    return pl.pallas_call(
        _mamba_ffn_kernel(L, T, di, ds, dc, dtr, dm, dff),
        grid=(Bz,),
        in_specs=[pl.BlockSpec((1, L, dm), lambda b: (b, 0, 0))] + wspec,
        out_specs=pl.BlockSpec((1, L, dm), lambda b: (b, 0, 0)),
        out_shape=jax.ShapeDtypeStruct((Bz, L, dm), f32),
        scratch_shapes=[
            pltpu.VMEM((L, di), f32), pltpu.VMEM((L, di), f32),
            pltpu.VMEM((L, ds), f32), pltpu.VMEM((L, ds), f32),
            pltpu.VMEM((L, di), f32),
            pltpu.VMEM((T, ds, di), f32), pltpu.VMEM((T, ds, di), f32),
            pltpu.VMEM((T, ds, 1), f32),
        ],
        compiler_params=pltpu.CompilerParams(
            dimension_semantics=('parallel',)),
    )(x, *ws)


def _tem_kernel(P, D):
    def body(x_ref, w0_ref, w1_ref, w2_ref, b_ref, g_ref, be_ref, o_ref):
        x = x_ref[0]
        zp = jnp.zeros((1, D), jnp.float32)
        xpad = jnp.concatenate([zp, x, zp], axis=0)
        c = b_ref[...]
        for k in range(3):
            c = c + jnp.dot(xpad[k:k + P, :], [w0_ref, w1_ref, w2_ref][k][...],
                            preferred_element_type=jnp.float32)
        c = _ln(c, g_ref[...], be_ref[...])
        o_ref[0] = c + x
    return body


def _tem(enc_pd, tp):
    Bz, N, P, D = enc_pd.shape
    x = enc_pd.reshape(Bz * N, P, D)
    ws = [tp['conv_w'][:, :, 0].T, tp['conv_w'][:, :, 1].T,
          tp['conv_w'][:, :, 2].T, tp['conv_b'].reshape(1, -1),
          tp['ln_g'].reshape(1, -1), tp['ln_b'].reshape(1, -1)]
    wspec = [pl.BlockSpec(w.shape, lambda b, n=w.ndim: (0,) * n) for w in ws]
    out = pl.pallas_call(
        _tem_kernel(P, D),
        grid=(Bz * N,),
        in_specs=[pl.BlockSpec((1, P, D), lambda b: (b, 0, 0))] + wspec,
        out_specs=pl.BlockSpec((1, P, D), lambda b: (b, 0, 0)),
        out_shape=jax.ShapeDtypeStruct((Bz * N, P, D), jnp.float32),
        compiler_params=pltpu.CompilerParams(
            dimension_semantics=('parallel',)),
    )(x, *ws)
    return out.reshape(Bz, N, P, D)


def kernel(x_enc, x_mark_enc, x_dec, x_mark_dec, params):
    p = params
    mean = jnp.mean(x_enc, 1, keepdims=True)
    std = jnp.sqrt(jnp.var(x_enc, 1, keepdims=True) + EPS)
    xn = (x_enc - mean) / std * p['revin_w'] + p['revin_b']

    emb = jnp.swapaxes(xn, 1, 2) @ p['emb_w'].T + p['emb_b']
    for lp in p['mamba1']:
        emb = _mamba_ffn(emb, lp)
    x_var = jnp.swapaxes(emb @ p['proj_w'].T + p['proj_b'], 1, 2)

    xp = jnp.swapaxes(xn, 1, 2)
    xp = jnp.concatenate(
        [xp, jnp.broadcast_to(xp[..., -1:], xp.shape[:-1] + (STRIDE,))], -1)
    idx = np.arange(PATCH_NUM)[:, None] * STRIDE + np.arange(PATCH_LEN)[None, :]
    patches = xp[..., idx]
    enc = patches @ p['val_w'].T + jnp.asarray(_POS)
    Bz, N, P, D = enc.shape
    for i in range(E_LAYERS):
        enc_t = _tem(enc, p['tem'][i])
        vp = p['var'][i]
        tokens = jnp.swapaxes(enc_t, -1, -2).reshape(Bz, P * N, D)
        v = _mamba_ffn(tokens, vp['mamba'])
        v = _ln(v.reshape(Bz, N, P, D), vp['ln_g'], vp['ln_b'])
        v = v + enc_t
        g, b = (p['ln2_g'], p['ln2_b']) if i == 0 else (p['ln3_g'], p['ln3_b'])
        enc = _ln(v, g, b)
    enc_dp = jnp.swapaxes(enc, -1, -2).reshape(Bz, N, D * P)
    x_patch = jnp.swapaxes(enc_dp @ p['head_w'].T + p['head_b'], 1, 2)

    out = x_var + x_patch
    out = (out - p['revin_b']) / (p['revin_w'] + EPS * EPS)
    return out * std + mean
